# baseline (speedup 1.0000x reference)
import functools

import jax
import jax.numpy as jnp
from jax import lax
from jax.experimental import pallas as pl
from jax.experimental.pallas import tpu as pltpu
from jax.experimental.pallas import tpu_sc as plsc

B = 10000
K = 32
S = 16
D = 128
E = 64
L = 16
NC = 2
NS = 16
NW = NC * NS
CHUNK = 312
LAST = B - (NW - 1) * CHUNK
G = 8
NG = CHUNK // G
NG_LAST = LAST // G


def _select16(bs_v, ns_v, nb_v, row, outs_b, idxn_b, j):
    iota = lax.iota(jnp.int32, L)
    center = plsc.load_gather(bs_v, [jnp.full((L,), 2 * row, jnp.int32)])
    sbase = jnp.full((L,), 2 * K * row, jnp.int32) + iota * 2
    sa = plsc.load_gather(ns_v, [sbase])
    sb = plsc.load_gather(ns_v, [sbase + K])
    da = jnp.abs(sa - center)
    db = jnp.abs(sb - center)
    ibase = jnp.full((L,), K * row, jnp.int32) + iota
    ia = plsc.load_gather(nb_v, [ibase])
    ib = plsc.load_gather(nb_v, [ibase + L])
    ka, va = plsc.sort_key_val(da, ia)
    kb, vb = plsc.sort_key_val(db, ib)
    rkb = lax.rev(kb, (0,))
    rvb = lax.rev(vb, (0,))
    ta = ka <= rkb
    km = jnp.where(ta, ka, rkb)
    vm = jnp.where(ta, va, rvb)
    ks, vs = plsc.sort_key_val(km, vm)
    outs_b[j, :] = ks
    idxn_b[pl.ds(j * L, L)] = vs


def _sc_body(bs_hbm, ns_hbm, nb_hbm, nd_hbm, feat_hbm, cat_hbm, samp_hbm,
             bs_v, ns_v, nb_v, nd_v,
             idxn0, idxn1, idxs0, idxs1,
             rown0, rown1, rows0, rows1,
             outc0, outc1, outs0, outs1,
             gsem0, gsem1, osem0, osem1):
    wid = lax.axis_index("s") * NC + lax.axis_index("c")
    base = wid * CHUNK
    ngroups = jnp.where(wid == NW - 1, NG_LAST, NG)

    pltpu.sync_copy(bs_hbm.at[pl.ds(base * 2, LAST * 2)], bs_v)
    pltpu.sync_copy(ns_hbm.at[pl.ds(base * 2 * K, LAST * 2 * K)], ns_v)
    pltpu.sync_copy(nb_hbm.at[pl.ds(base * K, LAST * K)], nb_v)
    pltpu.sync_copy(nd_hbm.at[pl.ds(base, LAST)], nd_v)

    idxn = (idxn0, idxn1)
    idxs = (idxs0, idxs1)
    rown = (rown0, rown1)
    rows = (rows0, rows1)
    outc = (outc0, outc1)
    outs = (outs0, outs1)
    gsem = (gsem0, gsem1)
    osem = (osem0, osem1)

    def select_group(g, b):
        row0 = g * G
        for j in range(G):
            _select16(bs_v, ns_v, nb_v, row0 + j, outs[b], idxn[b], j)
        iota = lax.iota(jnp.int32, L)
        sel = jnp.minimum(iota, G - 1) + row0
        idxs[b][:] = plsc.load_gather(nd_v, [sel])

    def accumulate_group(gp, b):
        rn = rown[b]
        rs = rows[b]
        for j in range(G):
            r0 = j * S
            for d in range(D // L):
                sl = pl.ds(d * L, L)
                t = [rn[r0 + s, sl] for s in range(S)]
                while len(t) > 1:
                    t = [t[i] + t[i + 1] for i in range(0, len(t), 2)]
                outc[b][j, pl.ds(d * L, L)] = rs[j, sl]
                outc[b][j, pl.ds(D + d * L, L)] = t[0]
        dst = base + gp * G
        pltpu.async_copy(outc[b], cat_hbm.at[pl.ds(dst, G)], osem[b])
        pltpu.async_copy(outs[b], samp_hbm.at[pl.ds(dst, G)], osem[b])

    def step(g, b):
        @pl.when(g >= 2)
        def _():
            pltpu.make_async_copy(outc[b], cat_hbm.at[pl.ds(0, G)], osem[b]).wait()
            pltpu.make_async_copy(outs[b], samp_hbm.at[pl.ds(0, G)], osem[b]).wait()

        @pl.when(g < ngroups)
        def _():
            select_group(g, b)
            pltpu.async_copy(feat_hbm.at[idxn[b]], rown[b], gsem[b])
            pltpu.async_copy(feat_hbm.at[idxs[b]], rows[b], gsem[b])

        @pl.when((g >= 1) & (g <= ngroups))
        def _():
            nb = 1 - b
            pltpu.make_async_copy(feat_hbm.at[idxn[nb]], rown[nb], gsem[nb]).wait()
            pltpu.make_async_copy(feat_hbm.at[idxs[nb]], rows[nb], gsem[nb]).wait()
            accumulate_group(g - 1, nb)

    def outer(t, carry):
        step(2 * t, 0)
        step(2 * t + 1, 1)
        return carry

    lax.fori_loop(0, (ngroups + 2) // 2, outer, 0)

    @pl.when(((ngroups - 1) % 2) == 0)
    def _():
        pltpu.make_async_copy(outc[0], cat_hbm.at[pl.ds(0, G)], osem[0]).wait()
        pltpu.make_async_copy(outs[0], samp_hbm.at[pl.ds(0, G)], osem[0]).wait()

    @pl.when(((ngroups - 1) % 2) == 1)
    def _():
        pltpu.make_async_copy(outc[1], cat_hbm.at[pl.ds(0, G)], osem[1]).wait()
        pltpu.make_async_copy(outs[1], samp_hbm.at[pl.ds(0, G)], osem[1]).wait()


@jax.jit
def _sc_stage(batch_scores, ns2, neighs, nodes, features):
    mesh = plsc.VectorSubcoreMesh(core_axis_name="c", subcore_axis_name="s")
    run = pl.kernel(
        _sc_body,
        out_type=(
            jax.ShapeDtypeStruct((B, 2 * D), jnp.float32),
            jax.ShapeDtypeStruct((B, S), jnp.float32),
        ),
        mesh=mesh,
        compiler_params=pltpu.CompilerParams(needs_layout_passes=False),
        scratch_types=[
            pltpu.VMEM((LAST * 2,), jnp.float32),
            pltpu.VMEM((LAST * 2 * K,), jnp.float32),
            pltpu.VMEM((LAST * K,), jnp.int32),
            pltpu.VMEM((LAST,), jnp.int32),
            pltpu.VMEM((G * S,), jnp.int32),
            pltpu.VMEM((G * S,), jnp.int32),
            pltpu.VMEM((L,), jnp.int32),
            pltpu.VMEM((L,), jnp.int32),
            pltpu.VMEM((G * S, D), jnp.float32),
            pltpu.VMEM((G * S, D), jnp.float32),
            pltpu.VMEM((L, D), jnp.float32),
            pltpu.VMEM((L, D), jnp.float32),
            pltpu.VMEM((G, 2 * D), jnp.float32),
            pltpu.VMEM((G, 2 * D), jnp.float32),
            pltpu.VMEM((G, S), jnp.float32),
            pltpu.VMEM((G, S), jnp.float32),
            pltpu.SemaphoreType.DMA,
            pltpu.SemaphoreType.DMA,
            pltpu.SemaphoreType.DMA,
            pltpu.SemaphoreType.DMA,
        ],
    )
    return run(batch_scores, ns2, neighs, nodes, features)


def _mm_body(cat_ref, w_ref, out_ref):
    out_ref[...] = jnp.maximum(
        jnp.dot(cat_ref[...], w_ref[...], preferred_element_type=jnp.float32),
        0.0,
    )


@jax.jit
def _tc_matmul(cat, w):
    bm = 1000
    return pl.pallas_call(
        _mm_body,
        grid=(B // bm,),
        in_specs=[
            pl.BlockSpec((bm, 2 * D), lambda i: (i, 0)),
            pl.BlockSpec((2 * D, E), lambda i: (0, 0)),
        ],
        out_specs=pl.BlockSpec((bm, E), lambda i: (i, 0)),
        out_shape=jax.ShapeDtypeStruct((B, E), jnp.float32),
    )(cat, w)


def kernel(nodes, neighs, batch_scores, neigh_scores, features, weight, num_sample):
    ns1 = neigh_scores.reshape(B * 2 * K)
    bs1 = batch_scores.reshape(B * 2)
    nb1 = neighs.astype(jnp.int32).reshape(B * K)
    cat, samp = _sc_stage(bs1, ns1, nb1, nodes.astype(jnp.int32), features)
    inv = 1.0 / jnp.asarray(num_sample, jnp.float32)
    w_scaled = jnp.concatenate([weight[:D], weight[D:] * inv], axis=0)
    to_feats = _tc_matmul(cat, w_scaled)
    return (to_feats, samp)


# speedup vs baseline: 1.8726x; 1.8726x over previous
import functools

import jax
import jax.numpy as jnp
from jax import lax
from jax.experimental import pallas as pl
from jax.experimental.pallas import tpu as pltpu
from jax.experimental.pallas import tpu_sc as plsc

B = 10000
K = 32
S = 16
D = 128
E = 64
L = 16
NC = 2
NS = 16
NW = NC * NS
CHUNK = 312
LAST = B - (NW - 1) * CHUNK
G = 8
NG = CHUNK // G
NG_LAST = LAST // G


def _select16(bs_v, ns_v, nb_v, row, outs_b, idxn_b, j):
    iota = lax.iota(jnp.int32, L)
    center = plsc.load_gather(bs_v, [jnp.full((L,), row, jnp.int32)])
    sbase = jnp.full((L,), K * row, jnp.int32) + iota
    sa = plsc.load_gather(ns_v, [sbase])
    sb = plsc.load_gather(ns_v, [sbase + L])
    da = jnp.abs(sa - center)
    db = jnp.abs(sb - center)
    ibase = jnp.full((L,), K * row, jnp.int32) + iota
    ia = plsc.load_gather(nb_v, [ibase])
    ib = plsc.load_gather(nb_v, [ibase + L])
    ka, va = plsc.sort_key_val(da, ia)
    kb, vb = plsc.sort_key_val(db, ib)
    rkb = lax.rev(kb, (0,))
    rvb = lax.rev(vb, (0,))
    ta = ka <= rkb
    km = jnp.where(ta, ka, rkb)
    vm = jnp.where(ta, va, rvb)
    ks, vs = plsc.sort_key_val(km, vm)
    outs_b[j, :] = ks
    idxn_b[pl.ds(j * L, L)] = vs


def _sc_body(bs_hbm, ns_hbm, nb_hbm, nd_hbm, feat_hbm, cat_hbm, samp_hbm,
             bs_v, ns_v, nb_v, nd_v,
             idxn0, idxn1, idxs0, idxs1,
             rown0, rown1, rows0, rows1,
             outc0, outc1, outs0, outs1,
             gsem0, gsem1, osem0, osem1):
    wid = lax.axis_index("s") * NC + lax.axis_index("c")
    base = wid * CHUNK
    ngroups = jnp.where(wid == NW - 1, NG_LAST, NG)

    pltpu.sync_copy(bs_hbm.at[pl.ds(base, LAST)], bs_v)
    pltpu.sync_copy(ns_hbm.at[pl.ds(base * K, LAST * K)], ns_v)
    pltpu.sync_copy(nb_hbm.at[pl.ds(base * K, LAST * K)], nb_v)
    pltpu.sync_copy(nd_hbm.at[pl.ds(base, LAST)], nd_v)

    idxn = (idxn0, idxn1)
    idxs = (idxs0, idxs1)
    rown = (rown0, rown1)
    rows = (rows0, rows1)
    outc = (outc0, outc1)
    outs = (outs0, outs1)
    gsem = (gsem0, gsem1)
    osem = (osem0, osem1)

    def select_group(g, b):
        row0 = g * G
        for j in range(G):
            _select16(bs_v, ns_v, nb_v, row0 + j, outs[b], idxn[b], j)
        iota = lax.iota(jnp.int32, L)
        sel = jnp.minimum(iota, G - 1) + row0
        idxs[b][:] = plsc.load_gather(nd_v, [sel])

    def accumulate_group(gp, b):
        rn = rown[b]
        rs = rows[b]
        for j in range(G):
            r0 = j * S
            for d in range(D // L):
                sl = pl.ds(d * L, L)
                t = [rn[r0 + s, sl] for s in range(S)]
                while len(t) > 1:
                    t = [t[i] + t[i + 1] for i in range(0, len(t), 2)]
                outc[b][j, pl.ds(d * L, L)] = rs[j, sl]
                outc[b][j, pl.ds(D + d * L, L)] = t[0]
        dst = base + gp * G
        pltpu.async_copy(outc[b], cat_hbm.at[pl.ds(dst, G)], osem[b])
        pltpu.async_copy(outs[b], samp_hbm.at[pl.ds(dst, G)], osem[b])

    def step(g, b):
        @pl.when(g >= 2)
        def _():
            pltpu.make_async_copy(outc[b], cat_hbm.at[pl.ds(0, G)], osem[b]).wait()
            pltpu.make_async_copy(outs[b], samp_hbm.at[pl.ds(0, G)], osem[b]).wait()

        @pl.when(g < ngroups)
        def _():
            select_group(g, b)
            pltpu.async_copy(feat_hbm.at[idxn[b]], rown[b], gsem[b])
            pltpu.async_copy(feat_hbm.at[idxs[b]], rows[b], gsem[b])

        @pl.when((g >= 1) & (g <= ngroups))
        def _():
            nb = 1 - b
            pltpu.make_async_copy(feat_hbm.at[idxn[nb]], rown[nb], gsem[nb]).wait()
            pltpu.make_async_copy(feat_hbm.at[idxs[nb]], rows[nb], gsem[nb]).wait()
            accumulate_group(g - 1, nb)

    def outer(t, carry):
        step(2 * t, 0)
        step(2 * t + 1, 1)
        return carry

    lax.fori_loop(0, (ngroups + 2) // 2, outer, 0)

    @pl.when(((ngroups - 1) % 2) == 0)
    def _():
        pltpu.make_async_copy(outc[0], cat_hbm.at[pl.ds(0, G)], osem[0]).wait()
        pltpu.make_async_copy(outs[0], samp_hbm.at[pl.ds(0, G)], osem[0]).wait()

    @pl.when(((ngroups - 1) % 2) == 1)
    def _():
        pltpu.make_async_copy(outc[1], cat_hbm.at[pl.ds(0, G)], osem[1]).wait()
        pltpu.make_async_copy(outs[1], samp_hbm.at[pl.ds(0, G)], osem[1]).wait()


@jax.jit
def _sc_stage(batch_scores, ns2, neighs, nodes, features):
    mesh = plsc.VectorSubcoreMesh(core_axis_name="c", subcore_axis_name="s")
    run = pl.kernel(
        _sc_body,
        out_type=(
            jax.ShapeDtypeStruct((B, 2 * D), jnp.float32),
            jax.ShapeDtypeStruct((B, S), jnp.float32),
        ),
        mesh=mesh,
        compiler_params=pltpu.CompilerParams(needs_layout_passes=False),
        scratch_types=[
            pltpu.VMEM((LAST,), jnp.float32),
            pltpu.VMEM((LAST * K,), jnp.float32),
            pltpu.VMEM((LAST * K,), jnp.int32),
            pltpu.VMEM((LAST,), jnp.int32),
            pltpu.VMEM((G * S,), jnp.int32),
            pltpu.VMEM((G * S,), jnp.int32),
            pltpu.VMEM((L,), jnp.int32),
            pltpu.VMEM((L,), jnp.int32),
            pltpu.VMEM((G * S, D), jnp.float32),
            pltpu.VMEM((G * S, D), jnp.float32),
            pltpu.VMEM((L, D), jnp.float32),
            pltpu.VMEM((L, D), jnp.float32),
            pltpu.VMEM((G, 2 * D), jnp.float32),
            pltpu.VMEM((G, 2 * D), jnp.float32),
            pltpu.VMEM((G, S), jnp.float32),
            pltpu.VMEM((G, S), jnp.float32),
            pltpu.SemaphoreType.DMA,
            pltpu.SemaphoreType.DMA,
            pltpu.SemaphoreType.DMA,
            pltpu.SemaphoreType.DMA,
        ],
    )
    return run(batch_scores, ns2, neighs, nodes, features)


def _mm_body(cat_ref, w_ref, out_ref):
    out_ref[...] = jnp.maximum(
        jnp.dot(cat_ref[...], w_ref[...], preferred_element_type=jnp.float32),
        0.0,
    )


@jax.jit
def _tc_matmul(cat, w):
    bm = 1000
    return pl.pallas_call(
        _mm_body,
        grid=(B // bm,),
        in_specs=[
            pl.BlockSpec((bm, 2 * D), lambda i: (i, 0)),
            pl.BlockSpec((2 * D, E), lambda i: (0, 0)),
        ],
        out_specs=pl.BlockSpec((bm, E), lambda i: (i, 0)),
        out_shape=jax.ShapeDtypeStruct((B, E), jnp.float32),
    )(cat, w)


def kernel(nodes, neighs, batch_scores, neigh_scores, features, weight, num_sample):
    ns1 = neigh_scores[:, :, 0].reshape(B * K)
    bs1 = batch_scores[:, 0]
    nb1 = neighs.astype(jnp.int32).reshape(B * K)
    cat, samp = _sc_stage(bs1, ns1, nb1, nodes.astype(jnp.int32), features)
    inv = 1.0 / jnp.asarray(num_sample, jnp.float32)
    w_scaled = jnp.concatenate([weight[:D], weight[D:] * inv], axis=0)
    to_feats = _tc_matmul(cat, w_scaled)
    return (to_feats, samp)


# speedup vs baseline: 3.2631x; 1.7426x over previous
import functools

import jax
import jax.numpy as jnp
from jax import lax
from jax.experimental import pallas as pl
from jax.experimental.pallas import tpu as pltpu
from jax.experimental.pallas import tpu_sc as plsc

B = 10000
K = 32
S = 16
D = 128
E = 64
L = 16
NC = 2
NS = 16
NW = NC * NS
CHUNK = 312
LAST = B - (NW - 1) * CHUNK
G = 8
NG = CHUNK // G
NG_LAST = LAST // G


def _select16(bs_v, ns_v, nb_v, row, outs_b, idxn_b, j):
    iota = lax.iota(jnp.int32, L)
    center = plsc.load_gather(bs_v, [jnp.full((L,), row, jnp.int32)])
    sbase = jnp.full((L,), K * row, jnp.int32) + iota
    sa = plsc.load_gather(ns_v, [sbase])
    sb = plsc.load_gather(ns_v, [sbase + L])
    da = jnp.abs(sa - center)
    db = jnp.abs(sb - center)
    ibase = jnp.full((L,), K * row, jnp.int32) + iota
    ia = plsc.load_gather(nb_v, [ibase])
    ib = plsc.load_gather(nb_v, [ibase + L])
    ka, va = plsc.sort_key_val(da, ia)
    kb, vb = plsc.sort_key_val(db, ib)
    rkb = lax.rev(kb, (0,))
    rvb = lax.rev(vb, (0,))
    ta = ka <= rkb
    km = jnp.where(ta, ka, rkb)
    vm = jnp.where(ta, va, rvb)
    ks, vs = plsc.sort_key_val(km, vm)
    outs_b[j, :] = ks
    idxn_b[pl.ds(j * L, L)] = vs


def _sc_body(bs_hbm, ns_hbm, nb_hbm, nd_hbm, feat_hbm, self_hbm, agg_hbm, samp_hbm,
             bs_v, ns_v, nb_v, nd_v,
             idxn0, idxn1, idxs0, idxs1,
             rown0, rown1, rows0, rows1,
             outa0, outa1, outs0, outs1,
             gsem0, gsem1, osem0, osem1):
    wid = lax.axis_index("s") * NC + lax.axis_index("c")
    base = wid * CHUNK
    ngroups = jnp.where(wid == NW - 1, NG_LAST, NG)

    pltpu.sync_copy(bs_hbm.at[pl.ds(base, LAST)], bs_v)
    pltpu.sync_copy(ns_hbm.at[pl.ds(base * K, LAST * K)], ns_v)
    pltpu.sync_copy(nb_hbm.at[pl.ds(base * K, LAST * K)], nb_v)
    pltpu.sync_copy(nd_hbm.at[pl.ds(base, LAST)], nd_v)

    idxn = (idxn0, idxn1)
    idxs = (idxs0, idxs1)
    rown = (rown0, rown1)
    rows = (rows0, rows1)
    outa = (outa0, outa1)
    outs = (outs0, outs1)
    gsem = (gsem0, gsem1)
    osem = (osem0, osem1)

    def select_group(g, b):
        row0 = g * G
        for j in range(G):
            _select16(bs_v, ns_v, nb_v, row0 + j, outs[b], idxn[b], j)
        iota = lax.iota(jnp.int32, L)
        sel = jnp.minimum(iota, G - 1) + row0
        idxs[b][:] = plsc.load_gather(nd_v, [sel])

    def accumulate_group(gp, b):
        rn = rown[b]
        oa = outa[b]

        @plsc.parallel_loop(0, G * (D // L), 1, unroll=2)
        def _(i):
            j = i >> 3
            d = i & 7
            r0 = j * S
            sl = pl.ds(d * L, L)
            p = [rn[r0 + s, sl] for s in range(4)]
            for s in range(4, S):
                p[s % 4] = p[s % 4] + rn[r0 + s, sl]
            oa[j, sl] = (p[0] + p[1]) + (p[2] + p[3])

        dst = base + gp * G
        pltpu.async_copy(rows[b].at[pl.ds(0, G)], self_hbm.at[pl.ds(dst, G)], osem[b])
        pltpu.async_copy(outa[b], agg_hbm.at[pl.ds(dst, G)], osem[b])
        pltpu.async_copy(outs[b], samp_hbm.at[pl.ds(dst, G)], osem[b])

    def step(g, b):
        @pl.when(g >= 2)
        def _():
            pltpu.make_async_copy(rows[b].at[pl.ds(0, G)], self_hbm.at[pl.ds(0, G)], osem[b]).wait()
            pltpu.make_async_copy(outa[b], agg_hbm.at[pl.ds(0, G)], osem[b]).wait()
            pltpu.make_async_copy(outs[b], samp_hbm.at[pl.ds(0, G)], osem[b]).wait()

        @pl.when(g < ngroups)
        def _():
            select_group(g, b)
            pltpu.async_copy(feat_hbm.at[idxn[b]], rown[b], gsem[b])
            pltpu.async_copy(feat_hbm.at[idxs[b]], rows[b], gsem[b])

        @pl.when((g >= 1) & (g <= ngroups))
        def _():
            nb = 1 - b
            pltpu.make_async_copy(feat_hbm.at[idxn[nb]], rown[nb], gsem[nb]).wait()
            pltpu.make_async_copy(feat_hbm.at[idxs[nb]], rows[nb], gsem[nb]).wait()
            accumulate_group(g - 1, nb)

    def outer(t, carry):
        step(2 * t, 0)
        step(2 * t + 1, 1)
        return carry

    lax.fori_loop(0, (ngroups + 2) // 2, outer, 0)

    @pl.when(((ngroups - 1) % 2) == 0)
    def _():
        pltpu.make_async_copy(rows[0].at[pl.ds(0, G)], self_hbm.at[pl.ds(0, G)], osem[0]).wait()
        pltpu.make_async_copy(outa[0], agg_hbm.at[pl.ds(0, G)], osem[0]).wait()
        pltpu.make_async_copy(outs[0], samp_hbm.at[pl.ds(0, G)], osem[0]).wait()

    @pl.when(((ngroups - 1) % 2) == 1)
    def _():
        pltpu.make_async_copy(rows[1].at[pl.ds(0, G)], self_hbm.at[pl.ds(0, G)], osem[1]).wait()
        pltpu.make_async_copy(outa[1], agg_hbm.at[pl.ds(0, G)], osem[1]).wait()
        pltpu.make_async_copy(outs[1], samp_hbm.at[pl.ds(0, G)], osem[1]).wait()


@jax.jit
def _sc_stage(batch_scores, ns2, neighs, nodes, features):
    mesh = plsc.VectorSubcoreMesh(core_axis_name="c", subcore_axis_name="s")
    run = pl.kernel(
        _sc_body,
        out_type=(
            jax.ShapeDtypeStruct((B, D), jnp.float32),
            jax.ShapeDtypeStruct((B, D), jnp.float32),
            jax.ShapeDtypeStruct((B, S), jnp.float32),
        ),
        mesh=mesh,
        compiler_params=pltpu.CompilerParams(needs_layout_passes=False),
        scratch_types=[
            pltpu.VMEM((LAST,), jnp.float32),
            pltpu.VMEM((LAST * K,), jnp.float32),
            pltpu.VMEM((LAST * K,), jnp.int32),
            pltpu.VMEM((LAST,), jnp.int32),
            pltpu.VMEM((G * S,), jnp.int32),
            pltpu.VMEM((G * S,), jnp.int32),
            pltpu.VMEM((L,), jnp.int32),
            pltpu.VMEM((L,), jnp.int32),
            pltpu.VMEM((G * S, D), jnp.float32),
            pltpu.VMEM((G * S, D), jnp.float32),
            pltpu.VMEM((L, D), jnp.float32),
            pltpu.VMEM((L, D), jnp.float32),
            pltpu.VMEM((G, D), jnp.float32),
            pltpu.VMEM((G, D), jnp.float32),
            pltpu.VMEM((G, S), jnp.float32),
            pltpu.VMEM((G, S), jnp.float32),
            pltpu.SemaphoreType.DMA,
            pltpu.SemaphoreType.DMA,
            pltpu.SemaphoreType.DMA,
            pltpu.SemaphoreType.DMA,
        ],
    )
    return run(batch_scores, ns2, neighs, nodes, features)


def _mm_body(sf_ref, ag_ref, wt_ref, wb_ref, out_ref):
    acc = jnp.dot(sf_ref[...], wt_ref[...], preferred_element_type=jnp.float32)
    acc += jnp.dot(ag_ref[...], wb_ref[...], preferred_element_type=jnp.float32)
    out_ref[...] = jnp.maximum(acc, 0.0)


@jax.jit
def _tc_matmul(sf, ag, wt, wb):
    bm = 1000
    return pl.pallas_call(
        _mm_body,
        grid=(B // bm,),
        in_specs=[
            pl.BlockSpec((bm, D), lambda i: (i, 0)),
            pl.BlockSpec((bm, D), lambda i: (i, 0)),
            pl.BlockSpec((D, E), lambda i: (0, 0)),
            pl.BlockSpec((D, E), lambda i: (0, 0)),
        ],
        out_specs=pl.BlockSpec((bm, E), lambda i: (i, 0)),
        out_shape=jax.ShapeDtypeStruct((B, E), jnp.float32),
    )(sf, ag, wt, wb)


def kernel(nodes, neighs, batch_scores, neigh_scores, features, weight, num_sample):
    ns1 = neigh_scores[:, :, 0].reshape(B * K)
    bs1 = batch_scores[:, 0]
    nb1 = neighs.astype(jnp.int32).reshape(B * K)
    sf, ag, samp = _sc_stage(bs1, ns1, nb1, nodes.astype(jnp.int32), features)
    inv = 1.0 / jnp.asarray(num_sample, jnp.float32)
    to_feats = _tc_matmul(sf, ag, weight[:D], weight[D:] * inv)
    return (to_feats, samp)


# speedup vs baseline: 3.5956x; 1.1019x over previous
import functools

import jax
import jax.numpy as jnp
from jax import lax
from jax.experimental import pallas as pl
from jax.experimental.pallas import tpu as pltpu
from jax.experimental.pallas import tpu_sc as plsc

B = 10000
K = 32
S = 16
D = 128
E = 64
L = 16
NC = 2
NS = 16
NW = NC * NS
CHUNK = 312
LAST = B - (NW - 1) * CHUNK
G = 8
NG = CHUNK // G
NG_LAST = LAST // G


def _select16(bs_v, ns_v, nb_v, row, outs_b, idxn_b, j):
    iota = lax.iota(jnp.int32, L)
    center = plsc.load_gather(bs_v, [jnp.full((L,), row, jnp.int32)])
    sbase = jnp.full((L,), K * row, jnp.int32) + iota
    sa = plsc.load_gather(ns_v, [sbase])
    sb = plsc.load_gather(ns_v, [sbase + L])
    da = jnp.abs(sa - center)
    db = jnp.abs(sb - center)
    ibase = jnp.full((L,), K * row, jnp.int32) + iota
    ia = plsc.load_gather(nb_v, [ibase])
    ib = plsc.load_gather(nb_v, [ibase + L])
    ka, va = plsc.sort_key_val(da, ia)
    kb, vb = plsc.sort_key_val(db, ib)
    rkb = lax.rev(kb, (0,))
    rvb = lax.rev(vb, (0,))
    ta = ka <= rkb
    km = jnp.where(ta, ka, rkb)
    vm = jnp.where(ta, va, rvb)
    ks, vs = plsc.sort_key_val(km, vm)
    outs_b[j, :] = ks
    idxn_b[pl.ds(j * L, L)] = vs


def _sc_body(bs_hbm, ns_hbm, nb_hbm, nd_hbm, feat_hbm, self_hbm, agg_hbm, samp_hbm,
             bs_v, ns_v, nb_v, nd_v, selfr_v,
             idxn0, idxn1,
             rown0, rown1,
             outa0, outa1, outs0, outs1,
             gsem0, gsem1, osem0, osem1, ssem):
    wid = lax.axis_index("s") * NC + lax.axis_index("c")
    base = wid * CHUNK
    ngroups = jnp.where(wid == NW - 1, NG_LAST, NG)

    pltpu.sync_copy(bs_hbm.at[pl.ds(base, LAST)], bs_v)
    pltpu.sync_copy(ns_hbm.at[pl.ds(base * K, LAST * K)], ns_v)
    pltpu.sync_copy(nb_hbm.at[pl.ds(base * K, LAST * K)], nb_v)
    pltpu.sync_copy(nd_hbm.at[pl.ds(base, LAST)], nd_v)

    pltpu.async_copy(feat_hbm.at[nd_v.at[pl.ds(0, 128)]], selfr_v.at[pl.ds(0, 128)], ssem)
    pltpu.async_copy(feat_hbm.at[nd_v.at[pl.ds(128, 128)]], selfr_v.at[pl.ds(128, 128)], ssem)
    pltpu.async_copy(feat_hbm.at[nd_v.at[pl.ds(256, LAST - 256)]], selfr_v.at[pl.ds(256, LAST - 256)], ssem)

    idxn = (idxn0, idxn1)
    rown = (rown0, rown1)
    outa = (outa0, outa1)
    outs = (outs0, outs1)
    gsem = (gsem0, gsem1)
    osem = (osem0, osem1)

    def select_group(g, b):
        row0 = g * G
        for j in range(G):
            _select16(bs_v, ns_v, nb_v, row0 + j, outs[b], idxn[b], j)

    def accumulate_group(gp, b):
        rn = rown[b]
        oa = outa[b]

        @plsc.parallel_loop(0, G * (D // L), 1, unroll=2)
        def _(i):
            j = i >> 3
            d = i & 7
            r0 = j * S
            sl = pl.ds(d * L, L)
            p = [rn[r0 + s, sl] for s in range(4)]
            for s in range(4, S):
                p[s % 4] = p[s % 4] + rn[r0 + s, sl]
            oa[j, sl] = (p[0] + p[1]) + (p[2] + p[3])

        dst = base + gp * G
        pltpu.async_copy(outa[b], agg_hbm.at[pl.ds(dst, G)], osem[b])
        pltpu.async_copy(outs[b], samp_hbm.at[pl.ds(dst, G)], osem[b])

    def step(g, b):
        @pl.when(g >= 2)
        def _():
            pltpu.make_async_copy(outa[b], agg_hbm.at[pl.ds(0, G)], osem[b]).wait()
            pltpu.make_async_copy(outs[b], samp_hbm.at[pl.ds(0, G)], osem[b]).wait()

        @pl.when(g < ngroups)
        def _():
            select_group(g, b)
            pltpu.async_copy(feat_hbm.at[idxn[b]], rown[b], gsem[b])

        @pl.when((g >= 1) & (g <= ngroups))
        def _():
            nb = 1 - b
            pltpu.make_async_copy(feat_hbm.at[idxn[nb]], rown[nb], gsem[nb]).wait()
            accumulate_group(g - 1, nb)

    def outer(t, carry):
        step(2 * t, 0)
        step(2 * t + 1, 1)
        return carry

    lax.fori_loop(0, (ngroups + 2) // 2, outer, 0)

    @pl.when(((ngroups - 1) % 2) == 0)
    def _():
        pltpu.make_async_copy(outa[0], agg_hbm.at[pl.ds(0, G)], osem[0]).wait()
        pltpu.make_async_copy(outs[0], samp_hbm.at[pl.ds(0, G)], osem[0]).wait()

    @pl.when(((ngroups - 1) % 2) == 1)
    def _():
        pltpu.make_async_copy(outa[1], agg_hbm.at[pl.ds(0, G)], osem[1]).wait()
        pltpu.make_async_copy(outs[1], samp_hbm.at[pl.ds(0, G)], osem[1]).wait()

    pltpu.make_async_copy(feat_hbm.at[nd_v.at[pl.ds(0, 128)]], selfr_v.at[pl.ds(0, 128)], ssem).wait()
    pltpu.make_async_copy(feat_hbm.at[nd_v.at[pl.ds(128, 128)]], selfr_v.at[pl.ds(128, 128)], ssem).wait()
    pltpu.make_async_copy(feat_hbm.at[nd_v.at[pl.ds(256, LAST - 256)]], selfr_v.at[pl.ds(256, LAST - 256)], ssem).wait()
    pltpu.sync_copy(selfr_v.at[pl.ds(0, CHUNK)], self_hbm.at[pl.ds(base, CHUNK)])

    @pl.when(wid == NW - 1)
    def _():
        pltpu.sync_copy(selfr_v.at[pl.ds(CHUNK, LAST - CHUNK)],
                        self_hbm.at[pl.ds(base + CHUNK, LAST - CHUNK)])


@jax.jit
def _sc_stage(batch_scores, ns2, neighs, nodes, features):
    mesh = plsc.VectorSubcoreMesh(core_axis_name="c", subcore_axis_name="s")
    run = pl.kernel(
        _sc_body,
        out_type=(
            jax.ShapeDtypeStruct((B, D), jnp.float32),
            jax.ShapeDtypeStruct((B, D), jnp.float32),
            jax.ShapeDtypeStruct((B, S), jnp.float32),
        ),
        mesh=mesh,
        compiler_params=pltpu.CompilerParams(needs_layout_passes=False),
        scratch_types=[
            pltpu.VMEM((LAST,), jnp.float32),
            pltpu.VMEM((LAST * K,), jnp.float32),
            pltpu.VMEM((LAST * K,), jnp.int32),
            pltpu.VMEM((LAST,), jnp.int32),
            pltpu.VMEM((LAST, D), jnp.float32),
            pltpu.VMEM((G * S,), jnp.int32),
            pltpu.VMEM((G * S,), jnp.int32),
            pltpu.VMEM((G * S, D), jnp.float32),
            pltpu.VMEM((G * S, D), jnp.float32),
            pltpu.VMEM((G, D), jnp.float32),
            pltpu.VMEM((G, D), jnp.float32),
            pltpu.VMEM((G, S), jnp.float32),
            pltpu.VMEM((G, S), jnp.float32),
            pltpu.SemaphoreType.DMA,
            pltpu.SemaphoreType.DMA,
            pltpu.SemaphoreType.DMA,
            pltpu.SemaphoreType.DMA,
            pltpu.SemaphoreType.DMA,
        ],
    )
    return run(batch_scores, ns2, neighs, nodes, features)


def _mm_body(sf_ref, ag_ref, wt_ref, wb_ref, out_ref):
    acc = jnp.dot(sf_ref[...], wt_ref[...], preferred_element_type=jnp.float32)
    acc += jnp.dot(ag_ref[...], wb_ref[...], preferred_element_type=jnp.float32)
    out_ref[...] = jnp.maximum(acc, 0.0)


@jax.jit
def _tc_matmul(sf, ag, wt, wb):
    bm = 1000
    return pl.pallas_call(
        _mm_body,
        grid=(B // bm,),
        in_specs=[
            pl.BlockSpec((bm, D), lambda i: (i, 0)),
            pl.BlockSpec((bm, D), lambda i: (i, 0)),
            pl.BlockSpec((D, E), lambda i: (0, 0)),
            pl.BlockSpec((D, E), lambda i: (0, 0)),
        ],
        out_specs=pl.BlockSpec((bm, E), lambda i: (i, 0)),
        out_shape=jax.ShapeDtypeStruct((B, E), jnp.float32),
    )(sf, ag, wt, wb)


def kernel(nodes, neighs, batch_scores, neigh_scores, features, weight, num_sample):
    ns1 = neigh_scores[:, :, 0].reshape(B * K)
    bs1 = batch_scores[:, 0]
    nb1 = neighs.astype(jnp.int32).reshape(B * K)
    sf, ag, samp = _sc_stage(bs1, ns1, nb1, nodes.astype(jnp.int32), features)
    inv = 1.0 / jnp.asarray(num_sample, jnp.float32)
    to_feats = _tc_matmul(sf, ag, weight[:D], weight[D:] * inv)
    return (to_feats, samp)
